# baseline (speedup 1.0000x reference)
import jax
import jax.numpy as jnp
from jax.experimental import pallas as pl
from jax.experimental.pallas import tpu as pltpu

_MARGIN = 0.2
_N = 4096
_D = 128
_BLK = 512
_CH = 512
_NCH = _N // _CH
_CPB = _BLK // _CH
_NUM_LABELS = 64


def _triplet_kernel(e_blk_ref, e_full_ref, t_col_ref, t_row_ref, out_ref,
                    acc_ref, sqrow_ref):
    i = pl.program_id(0)
    nsteps = pl.num_programs(0)

    @pl.when(i == 0)
    def _init():
        acc_ref[0, 0] = 0.0
        acc_ref[0, 1] = 0.0
        sqrow_ref[...] = jax.lax.dot_general(
            jnp.ones((1, _D), jnp.float32),
            e_full_ref[...] * e_full_ref[...],
            (((1,), (1,)), ((), ())),
            precision=jax.lax.Precision.DEFAULT,
            preferred_element_type=jnp.float32)

    e_blk = e_blk_ref[...] * (-2.0)
    e_blk16 = e_blk.astype(jnp.bfloat16)
    t_col = t_col_ref[...]

    def g_chunk(jc):
        e_ch = e_full_ref[jc * _CH:(jc + 1) * _CH, :]
        mm = jax.lax.dot_general(
            e_blk16, e_ch.astype(jnp.bfloat16), (((1,), (1,)), ((), ())),
            precision=jax.lax.Precision.DEFAULT,
            preferred_element_type=jnp.float32)
        return mm + sqrow_ref[:, jc * _CH:(jc + 1) * _CH]

    rmin = jnp.full((_BLK, 1), jnp.inf, jnp.float32)
    for jc in range(_NCH):
        g = g_chunk(jc)
        same = t_col == t_row_ref[:, jc * _CH:(jc + 1) * _CH]
        masked = jnp.where(same, jnp.inf, g)
        rmin = jnp.minimum(rmin, jnp.min(masked, axis=1, keepdims=True))

    g0 = jnp.sum(e_blk * e_full_ref[0:1, :], axis=1,
                 keepdims=True) + sqrow_ref[0, 0]
    adj = jnp.where(jnp.isinf(rmin), g0, rmin) - _MARGIN

    rows = i * _BLK + jax.lax.broadcasted_iota(jnp.int32, (_BLK, 1), 0)

    for jc in range(_NCH):
        @pl.when(jc + 1 > i * _CPB)
        def _loss_chunk(jc=jc):
            g = g_chunk(jc)
            same = t_col == t_row_ref[:, jc * _CH:(jc + 1) * _CH]
            mv = g - adj
            cols = jc * _CH + jax.lax.broadcasted_iota(jnp.int32, (1, _CH), 1)
            validm = (same & (cols > rows)) & (mv > 0)
            acc_ref[0, 0] += jnp.sum(jnp.where(validm, mv, 0.0))
            acc_ref[0, 1] += jnp.sum(validm.astype(jnp.float32))

    @pl.when(i == nsteps - 1)
    def _finalize():
        total = acc_ref[0, 0]
        count = acc_ref[0, 1]
        trow = t_row_ref[...]
        big = jnp.int32(_N)
        lab_col = jax.lax.broadcasted_iota(jnp.int32, (_NUM_LABELS, 1), 0)
        iota = jax.lax.broadcasted_iota(jnp.int32, (_NUM_LABELS, _N), 1)
        msk = lab_col == trow
        cnt2 = jnp.sum(msk.astype(jnp.int32), axis=1, keepdims=True)
        i0 = jnp.min(jnp.where(msk, iota, big), axis=1, keepdims=True)
        m2 = msk & (iota != i0)
        i1 = jnp.min(jnp.where(m2, iota, big), axis=1, keepdims=True)
        n0 = jnp.min(jnp.where(jnp.logical_not(msk), iota, big),
                     axis=1, keepdims=True)
        i0 = jnp.where(i0 >= big, 0, i0)
        i1 = jnp.where(i1 >= big, 0, i1)
        n0 = jnp.where(n0 >= big, 0, n0)
        ok = cnt2 >= 2
        labsel = jnp.max(jnp.where(ok, lab_col, -1))
        sel = lab_col == labsel
        fb0 = jnp.max(jnp.where(sel, i0, 0))
        fb1 = jnp.max(jnp.where(sel, i1, 0))
        fb2 = jnp.max(jnp.where(sel, n0, 0))
        a = e_full_ref[pl.ds(fb0, 1), :]
        p = e_full_ref[pl.ds(fb1, 1), :]
        ng = e_full_ref[pl.ds(fb2, 1), :]
        fb_loss = jnp.maximum(
            jnp.sum((a - p) ** 2) - jnp.sum((a - ng) ** 2) + _MARGIN, 0.0)
        out_ref[0, 0] = jnp.where(count > 0, total / count, fb_loss)


def kernel(embeddings, target):
    t32 = target.astype(jnp.int32)
    t_col = t32.reshape(_N, 1)
    t_row = t32.reshape(1, _N)
    out = pl.pallas_call(
        _triplet_kernel,
        grid=(_N // _BLK,),
        in_specs=[
            pl.BlockSpec((_BLK, _D), lambda i: (i, 0)),
            pl.BlockSpec((_N, _D), lambda i: (0, 0)),
            pl.BlockSpec((_BLK, 1), lambda i: (i, 0)),
            pl.BlockSpec((1, _N), lambda i: (0, 0)),
        ],
        out_specs=pl.BlockSpec(memory_space=pltpu.SMEM),
        out_shape=jax.ShapeDtypeStruct((1, 1), jnp.float32),
        scratch_shapes=[pltpu.SMEM((1, 2), jnp.float32),
                        pltpu.VMEM((1, _N), jnp.float32)],
    )(embeddings, embeddings, t_col, t_row)
    m = out[0, 0]
    return (m, m)


# speedup vs baseline: 1.1182x; 1.1182x over previous
import jax
import jax.numpy as jnp
from jax.experimental import pallas as pl
from jax.experimental.pallas import tpu as pltpu

_MARGIN = 0.2
_N = 4096
_D = 128
_BLK = 1024
_CH = 1024
_NCH = _N // _CH
_CPB = _BLK // _CH
_NUM_LABELS = 64


def _triplet_kernel(e_blk_ref, e_full_ref, t_col_ref, t_row_ref, out_ref,
                    acc_ref, sqrow_ref):
    i = pl.program_id(0)
    nsteps = pl.num_programs(0)

    @pl.when(i == 0)
    def _init():
        acc_ref[0, 0] = 0.0
        acc_ref[0, 1] = 0.0
        sqrow_ref[...] = jax.lax.dot_general(
            jnp.ones((1, _D), jnp.float32),
            e_full_ref[...] * e_full_ref[...],
            (((1,), (1,)), ((), ())),
            precision=jax.lax.Precision.DEFAULT,
            preferred_element_type=jnp.float32)

    e_blk = e_blk_ref[...] * (-2.0)
    e_blk16 = e_blk.astype(jnp.bfloat16)
    t_col = t_col_ref[...]

    def g_chunk(jc):
        e_ch = e_full_ref[jc * _CH:(jc + 1) * _CH, :]
        mm = jax.lax.dot_general(
            e_blk16, e_ch.astype(jnp.bfloat16), (((1,), (1,)), ((), ())),
            precision=jax.lax.Precision.DEFAULT,
            preferred_element_type=jnp.float32)
        return mm + sqrow_ref[:, jc * _CH:(jc + 1) * _CH]

    rmin = jnp.full((_BLK, 1), jnp.inf, jnp.float32)
    for jc in range(_NCH):
        g = g_chunk(jc)
        same = t_col == t_row_ref[:, jc * _CH:(jc + 1) * _CH]
        masked = jnp.where(same, jnp.inf, g)
        rmin = jnp.minimum(rmin, jnp.min(masked, axis=1, keepdims=True))

    g0 = jnp.sum(e_blk * e_full_ref[0:1, :], axis=1,
                 keepdims=True) + sqrow_ref[0, 0]
    adj = jnp.where(jnp.isinf(rmin), g0, rmin) - _MARGIN

    rows = i * _BLK + jax.lax.broadcasted_iota(jnp.int32, (_BLK, 1), 0)

    for jc in range(_NCH):
        @pl.when(jc + 1 > i * _CPB)
        def _loss_chunk(jc=jc):
            g = g_chunk(jc)
            same = t_col == t_row_ref[:, jc * _CH:(jc + 1) * _CH]
            mv = g - adj
            cols = jc * _CH + jax.lax.broadcasted_iota(jnp.int32, (1, _CH), 1)
            validm = (same & (cols > rows)) & (mv > 0)
            acc_ref[0, 0] += jnp.sum(jnp.where(validm, mv, 0.0))
            acc_ref[0, 1] += jnp.sum(validm.astype(jnp.float32))

    @pl.when(i == nsteps - 1)
    def _finalize():
        total = acc_ref[0, 0]
        count = acc_ref[0, 1]
        trow = t_row_ref[...]
        big = jnp.int32(_N)
        lab_col = jax.lax.broadcasted_iota(jnp.int32, (_NUM_LABELS, 1), 0)
        iota = jax.lax.broadcasted_iota(jnp.int32, (_NUM_LABELS, _N), 1)
        msk = lab_col == trow
        cnt2 = jnp.sum(msk.astype(jnp.int32), axis=1, keepdims=True)
        i0 = jnp.min(jnp.where(msk, iota, big), axis=1, keepdims=True)
        m2 = msk & (iota != i0)
        i1 = jnp.min(jnp.where(m2, iota, big), axis=1, keepdims=True)
        n0 = jnp.min(jnp.where(jnp.logical_not(msk), iota, big),
                     axis=1, keepdims=True)
        i0 = jnp.where(i0 >= big, 0, i0)
        i1 = jnp.where(i1 >= big, 0, i1)
        n0 = jnp.where(n0 >= big, 0, n0)
        ok = cnt2 >= 2
        labsel = jnp.max(jnp.where(ok, lab_col, -1))
        sel = lab_col == labsel
        fb0 = jnp.max(jnp.where(sel, i0, 0))
        fb1 = jnp.max(jnp.where(sel, i1, 0))
        fb2 = jnp.max(jnp.where(sel, n0, 0))
        a = e_full_ref[pl.ds(fb0, 1), :]
        p = e_full_ref[pl.ds(fb1, 1), :]
        ng = e_full_ref[pl.ds(fb2, 1), :]
        fb_loss = jnp.maximum(
            jnp.sum((a - p) ** 2) - jnp.sum((a - ng) ** 2) + _MARGIN, 0.0)
        out_ref[0, 0] = jnp.where(count > 0, total / count, fb_loss)


def kernel(embeddings, target):
    t32 = target.astype(jnp.int32)
    t_col = t32.reshape(_N, 1)
    t_row = t32.reshape(1, _N)
    out = pl.pallas_call(
        _triplet_kernel,
        grid=(_N // _BLK,),
        in_specs=[
            pl.BlockSpec((_BLK, _D), lambda i: (i, 0)),
            pl.BlockSpec((_N, _D), lambda i: (0, 0)),
            pl.BlockSpec((_BLK, 1), lambda i: (i, 0)),
            pl.BlockSpec((1, _N), lambda i: (0, 0)),
        ],
        out_specs=pl.BlockSpec(memory_space=pltpu.SMEM),
        out_shape=jax.ShapeDtypeStruct((1, 1), jnp.float32),
        scratch_shapes=[pltpu.SMEM((1, 2), jnp.float32),
                        pltpu.VMEM((1, _N), jnp.float32)],
    )(embeddings, embeddings, t_col, t_row)
    m = out[0, 0]
    return (m, m)
